# baseline (speedup 1.0000x reference)
import functools

import jax
import jax.numpy as jnp
from jax import lax
from jax.experimental import pallas as pl
from jax.experimental.pallas import tpu as pltpu


def _se_stream_kernel(x_hbm, w1t_ref, w2t_ref, o_hbm,
                      xbuf, obuf, in_sems, out_sems,
                      *, n_chunks, bc, din, dout, inv_hw):
    def start_in(i, slot):
        pltpu.make_async_copy(
            x_hbm.at[pl.ds(i * bc, bc)], xbuf.at[slot], in_sems.at[slot]
        ).start()

    def wait_in(slot):
        pltpu.make_async_copy(
            xbuf.at[slot], xbuf.at[slot], in_sems.at[slot]
        ).wait()

    def start_out(i, slot):
        pltpu.make_async_copy(
            obuf.at[slot], o_hbm.at[pl.ds(i * bc, bc)], out_sems.at[slot]
        ).start()

    def wait_out(slot):
        pltpu.make_async_copy(
            obuf.at[slot], obuf.at[slot], out_sems.at[slot]
        ).wait()

    for j in range(min(din - 1, n_chunks)):
        start_in(j, j)

    w1t = w1t_ref[...]
    w2t = w2t_ref[...]

    def body(i, _):
        si = lax.rem(i, din)
        so = lax.rem(i, dout)

        nxt = i + din - 1

        @pl.when(nxt < n_chunks)
        def _():
            start_in(nxt, lax.rem(nxt, din))

        wait_in(si)
        x = xbuf[si]

        pooled = jnp.sum(x, axis=2, dtype=jnp.float32) * inv_hw
        h = jnp.dot(pooled, w1t, preferred_element_type=jnp.float32)
        h = jnp.maximum(h, 0.0)
        g = jnp.dot(h, w2t, preferred_element_type=jnp.float32)
        g = jax.nn.sigmoid(g)

        @pl.when(i >= dout)
        def _():
            wait_out(so)

        obuf[so] = x * g[:, :, None]
        start_out(i, so)
        return ()

    lax.fori_loop(0, n_chunks, body, (), unroll=False)

    for j in range(max(n_chunks - dout, 0), n_chunks):
        wait_out(j % dout)


def kernel(x, w_fc1, w_fc2):
    B, C, H, W = x.shape
    HW = H * W
    Cr = w_fc1.shape[0]
    x_flat = x.reshape(B, C, HW)
    w1t = w_fc1.T
    w2t = w_fc2.T

    bc = 2 if B % 2 == 0 else 1
    n_chunks = B // bc
    din = min(5, n_chunks + 1)
    dout = min(4, n_chunks)

    out_flat = pl.pallas_call(
        functools.partial(_se_stream_kernel, n_chunks=n_chunks, bc=bc,
                          din=din, dout=dout, inv_hw=1.0 / HW),
        out_shape=jax.ShapeDtypeStruct((B, C, HW), jnp.float32),
        in_specs=[
            pl.BlockSpec(memory_space=pl.ANY),
            pl.BlockSpec(memory_space=pltpu.VMEM),
            pl.BlockSpec(memory_space=pltpu.VMEM),
        ],
        out_specs=pl.BlockSpec(memory_space=pl.ANY),
        scratch_shapes=[
            pltpu.VMEM((din, bc, C, HW), jnp.float32),
            pltpu.VMEM((dout, bc, C, HW), jnp.float32),
            pltpu.SemaphoreType.DMA((din,)),
            pltpu.SemaphoreType.DMA((dout,)),
        ],
        compiler_params=pltpu.CompilerParams(
            vmem_limit_bytes=60 << 20,
        ),
    )(x_flat, w1t, w2t)
    return out_flat.reshape(B, C, H, W)


# speedup vs baseline: 1.0007x; 1.0007x over previous
import functools

import jax
import jax.numpy as jnp
from jax import lax
from jax.experimental import pallas as pl
from jax.experimental.pallas import tpu as pltpu


def _se_stream_kernel(x_hbm, w1t_ref, w2t_ref, o_hbm,
                      xbuf, obuf, in_sems, out_sems,
                      *, n_chunks, bc, din, dout, inv_hw):
    def start_in(i, slot):
        pltpu.make_async_copy(
            x_hbm.at[pl.ds(i * bc, bc)], xbuf.at[slot], in_sems.at[slot]
        ).start()

    def wait_in(slot):
        pltpu.make_async_copy(
            xbuf.at[slot], xbuf.at[slot], in_sems.at[slot]
        ).wait()

    def start_out(i, slot):
        pltpu.make_async_copy(
            obuf.at[slot], o_hbm.at[pl.ds(i * bc, bc)], out_sems.at[slot]
        ).start(priority=1)

    def wait_out(slot):
        pltpu.make_async_copy(
            obuf.at[slot], obuf.at[slot], out_sems.at[slot]
        ).wait()

    for j in range(min(din - 1, n_chunks)):
        start_in(j, j)

    w1t = w1t_ref[...]
    w2t = w2t_ref[...]

    def body(i, _):
        si = lax.rem(i, din)
        so = lax.rem(i, dout)

        nxt = i + din - 1

        @pl.when(nxt < n_chunks)
        def _():
            start_in(nxt, lax.rem(nxt, din))

        wait_in(si)
        x = xbuf[si]

        pooled = jnp.sum(x, axis=2, dtype=jnp.float32) * inv_hw
        h = jnp.dot(pooled, w1t, preferred_element_type=jnp.float32)
        h = jnp.maximum(h, 0.0)
        g = jnp.dot(h, w2t, preferred_element_type=jnp.float32)
        g = jax.nn.sigmoid(g)

        @pl.when(i >= dout)
        def _():
            wait_out(so)

        obuf[so] = x * g[:, :, None]
        start_out(i, so)
        return ()

    lax.fori_loop(0, n_chunks, body, (), unroll=False)

    for j in range(max(n_chunks - dout, 0), n_chunks):
        wait_out(j % dout)


def kernel(x, w_fc1, w_fc2):
    B, C, H, W = x.shape
    HW = H * W
    Cr = w_fc1.shape[0]
    x_flat = x.reshape(B, C, HW)
    w1t = w_fc1.T
    w2t = w_fc2.T

    bc = 2 if B % 2 == 0 else 1
    n_chunks = B // bc
    din = min(5, n_chunks + 1)
    dout = min(4, n_chunks)

    out_flat = pl.pallas_call(
        functools.partial(_se_stream_kernel, n_chunks=n_chunks, bc=bc,
                          din=din, dout=dout, inv_hw=1.0 / HW),
        out_shape=jax.ShapeDtypeStruct((B, C, HW), jnp.float32),
        in_specs=[
            pl.BlockSpec(memory_space=pl.ANY),
            pl.BlockSpec(memory_space=pltpu.VMEM),
            pl.BlockSpec(memory_space=pltpu.VMEM),
        ],
        out_specs=pl.BlockSpec(memory_space=pl.ANY),
        scratch_shapes=[
            pltpu.VMEM((din, bc, C, HW), jnp.float32),
            pltpu.VMEM((dout, bc, C, HW), jnp.float32),
            pltpu.SemaphoreType.DMA((din,)),
            pltpu.SemaphoreType.DMA((dout,)),
        ],
        compiler_params=pltpu.CompilerParams(
            vmem_limit_bytes=60 << 20,
        ),
    )(x_flat, w1t, w2t)
    return out_flat.reshape(B, C, H, W)
